# baseline (speedup 1.0000x reference)
import functools

import jax
import jax.numpy as jnp
from jax.experimental import pallas as pl
from jax.experimental.pallas import tpu as pltpu

CT = 8


def _fused_body(Bb, Hh, T, nc,
                s_ref,
                x_ref, code0_ref, code1_ref, idxS_ref, idxO_ref, nS_ref, nO_ref,
                wi1, wh1, bi1, bh1, wi2, wh2, bi2, bh2,
                wis1, whs1, bis1, bhs1, wis2, whs2, bis2, bhs2,
                wio1, who1, bio1, bho1, wio2, who2, bio2, bho2,
                fcw, fcb,
                out_ref,
                g1, g2, g3, g4, gSO, y2,
                y1, ys1, yo1,
                h1, h2, hs1, hs2, ho1, ho2, any_s, any_o):
    c = pl.program_id(0)
    f32 = jnp.float32
    bf16 = jnp.bfloat16
    ncA = s_ref[0]
    ncB = s_ref[1]
    p = jax.lax.rem(c, 2)
    q = 1 - p
    cb = c - (nc + 1)

    @pl.when(c == 0)
    def _init():
        for r in (h1, h2, hs1, hs2, ho1, ho2, any_s, any_o, y1, ys1, yo1, y2):
            r[...] = jnp.zeros_like(r)

    def dense(src, w_ref, b_ref, dst_ref):
        Xm = src.reshape(CT * Bb, -1).astype(bf16)
        dst_ref[...] = (
            jnp.dot(Xm, w_ref[...], preferred_element_type=f32) + b_ref[0:1, :]
        ).reshape(CT, Bb, 3 * Hh)

    def cell(gi, gh, h, bhn):
        r = jax.nn.sigmoid(gi[:, :Hh] + gh[:, :Hh])
        z = jax.nn.sigmoid(gi[:, Hh:2 * Hh] + gh[:, Hh:2 * Hh])
        n = jnp.tanh(gi[:, 2 * Hh:] + r * (gh[:, 2 * Hh:] + bhn))
        return (1.0 - z) * n + z * h

    def chain(gi_ref, t, h_ref, w_ref, b_ref, m):
        h = h_ref[...]
        gh = jnp.dot(h.astype(bf16), w_ref[...], preferred_element_type=f32)
        hv = jnp.where(m, cell(gi_ref[t], gh, h, b_ref[0:1, 2 * Hh:]), h)
        h_ref[...] = hv
        return hv

    @pl.when(c <= ncA)
    def _phase_a():
        dense(x_ref[...], wi1, bi1, g1)
        dense(y1[q], wi2, bi2, g2)
        a0 = c < ncA
        a1 = (c >= 1) & (c <= ncA)

        def step(t, carry):
            c0 = code0_ref[t]
            c1 = code1_ref[t]
            y1[p, t] = chain(g1, t, h1, wh1, bh1, (c0 != 0.0) & a0)
            hv2 = chain(g2, t, h2, wh2, bh2, (c1 != 0.0) & a1)
            tg = jnp.maximum((c - 1) * CT + t, 0)
            y2[pl.ds(tg * Bb, Bb), :] = hv2
            return carry

        jax.lax.fori_loop(0, CT, step, 0, unroll=16)

        codes = code0_ref[...]
        any_s[...] = jnp.maximum(any_s[...], jnp.max((codes > 0.0).astype(f32), axis=0))
        any_o[...] = jnp.maximum(any_o[...], jnp.max((codes < 0.0).astype(f32), axis=0))

    @pl.when((cb >= 0) & (cb < ncB))
    def _gather():
        iota_b = jax.lax.broadcasted_iota(jnp.int32, (CT, Bb), 1)
        targ = jnp.concatenate(
            [idxS_ref[...] * Bb + iota_b, idxO_ref[...] * Bb + iota_b], axis=0)
        iota_col = jax.lax.broadcasted_iota(jnp.int32, (1, 1, T * Bb), 2)
        p_all = (targ[:, :, None] == iota_col).astype(bf16).reshape(
            2 * CT * Bb, T * Bb)
        res = jnp.dot(p_all.astype(f32), y2[...], preferred_element_type=f32)
        gSO[...] = res.reshape(2, CT, Bb, Hh).astype(bf16)

    @pl.when((cb >= 0) & (cb <= ncB))
    def _phase_b():
        dense(gSO[0], wis1, bis1, g1)
        dense(gSO[1], wio1, bio1, g2)
        dense(ys1[q], wis2, bis2, g3)
        dense(yo1[q], wio2, bio2, g4)
        aL1 = cb < ncB
        aL2 = (cb >= 1) & (cb <= ncB)
        nS = nS_ref[...]
        nO = nO_ref[...]

        def step(t, carry):
            jg = cb * CT + t
            j2 = jg - CT
            jgf = jg.astype(f32)
            j2f = j2.astype(f32)
            ys1[p, t] = chain(g1, t, hs1, whs1, bhs1, (nS > jgf) & aL1)
            yo1[p, t] = chain(g2, t, ho1, who1, bho1, (nO > jgf) & aL1)
            chain(g3, t, hs2, whs2, bhs2, (nS > j2f) & aL2)
            chain(g4, t, ho2, who2, bho2, (nO > j2f) & aL2)
            return carry

        jax.lax.fori_loop(0, CT, step, 0, unroll=16)

    @pl.when(c == 2 * nc + 1)
    def _final():
        zero1 = jnp.zeros((1, Hh), f32)

        zero3 = jnp.zeros((1, 3 * Hh), f32)

        def fall2(bi_1, bh_1, wi_2, bi_2, bh_2):
            f1 = cell(bi_1[0:1, :], zero3, zero1, bh_1[0:1, 2 * Hh:])
            gi = jnp.dot(f1.astype(bf16), wi_2[...], preferred_element_type=f32) + bi_2[0:1, :]
            return cell(gi, zero3, zero1, bh_2[0:1, 2 * Hh:])

        fs = fall2(bis1, bhs1, wis2, bis2, bhs2)
        fo = fall2(bio1, bho1, wio2, bio2, bho2)
        hS = jnp.where(any_s[...] > 0.0, hs2[...], fs)
        hO = jnp.where(any_o[...] > 0.0, ho2[...], fo)
        hcat = jnp.concatenate([hS, hO, h2[...]], axis=1)
        out_ref[...] = jnp.dot(hcat, fcw[...], preferred_element_type=f32) + fcb[...]


def kernel(context_features, params_inter, params_spk, params_oth, fc_w, fc_b,
           context_lengths, context_speaker_ids, roles):
    f32 = jnp.float32
    bf16 = jnp.bfloat16
    Bb, T, D = context_features.shape
    Hh = params_inter[0][1].shape[1]
    C = fc_w.shape[0]
    nc = T // CT

    x = jnp.transpose(context_features, (1, 0, 2)).astype(bf16)

    lengths = jnp.asarray(context_lengths)
    sid = jnp.asarray(context_speaker_ids)
    roles_a = jnp.asarray(roles)
    t_idx = jnp.arange(T)
    valid = t_idx[:, None] < lengths[None, :]
    match = sid.T == roles_a[None, :]
    spk = valid & match
    oth = valid & (~match)
    code = jnp.where(valid, jnp.where(match, 1.0, -1.0), 0.0).astype(bf16)
    code_b = jnp.broadcast_to(code[:, :, None], (T, Bb, Hh))

    nS = jnp.sum(spk, axis=0)
    nO = jnp.sum(oth, axis=0)
    idxS = jnp.argsort(~spk, axis=0, stable=True).astype(jnp.int32)
    idxO = jnp.argsort(~oth, axis=0, stable=True).astype(jnp.int32)
    maxL = jnp.max(lengths)
    maxSub = jnp.maximum(jnp.max(nS), jnp.max(nO))
    ncA = jnp.clip((maxL + CT - 1) // CT, 1, nc).astype(jnp.int32)
    ncB = jnp.clip((maxSub + CT - 1) // CT, 1, nc).astype(jnp.int32)
    scalars = jnp.stack([ncA, ncB])
    nS_b = jnp.broadcast_to(nS.astype(f32)[:, None], (Bb, Hh))
    nO_b = jnp.broadcast_to(nO.astype(f32)[:, None], (Bb, Hh))

    def prep(pr):
        W_ih, W_hh, b_ih, b_hh = pr
        bi_fold = (b_ih + jnp.concatenate(
            [b_hh[:2 * Hh], jnp.zeros((Hh,), b_hh.dtype)])).astype(f32)
        return (W_ih.T.astype(bf16), W_hh.T.astype(bf16),
                jnp.broadcast_to(bi_fold[None, :], (Bb, 3 * Hh)),
                jnp.broadcast_to(b_hh[None, :].astype(f32), (Bb, 3 * Hh)))

    layers = [prep(pr) for pr in (params_inter + params_spk + params_oth)]
    w_args = [a for lay in layers for a in lay]

    fcw_pad = jnp.zeros((3 * Hh, 128), f32).at[:, :C].set(fc_w.T.astype(f32))
    fcb_pad = jnp.broadcast_to(
        jnp.zeros((128,), f32).at[:C].set(fc_b.astype(f32))[None, :], (Bb, 128))

    def a_spec(k, shape):
        return pl.BlockSpec(
            shape,
            lambda c, s, k=k: (jnp.clip(c - k, 0, jnp.minimum(s[0], nc - 1)), 0, 0))

    def b_spec(shape):
        return pl.BlockSpec(
            shape,
            lambda c, s: (jnp.clip(c - (nc + 1), 0, jnp.minimum(s[1], nc - 1)), 0))

    full2d = lambda a: pl.BlockSpec(a.shape, lambda c, s: (0, 0))
    in_specs = [
        a_spec(0, (CT, Bb, D)),
        a_spec(0, (CT, Bb, Hh)), a_spec(1, (CT, Bb, Hh)),
        b_spec((CT, Bb)), b_spec((CT, Bb)),
        full2d(nS_b), full2d(nO_b),
    ] + [full2d(a) for a in w_args] + [full2d(fcw_pad), full2d(fcb_pad)]

    scratch = (
        [pltpu.VMEM((CT, Bb, 3 * Hh), f32)] * 4
        + [pltpu.VMEM((2, CT, Bb, Hh), bf16)]
        + [pltpu.VMEM((T * Bb, Hh), f32)]
        + [pltpu.VMEM((2, CT, Bb, Hh), f32)] * 3
        + [pltpu.VMEM((Bb, Hh), f32)] * 8
    )

    body = functools.partial(_fused_body, Bb, Hh, T, nc)

    grid_spec = pltpu.PrefetchScalarGridSpec(
        num_scalar_prefetch=1,
        grid=(2 * nc + 2,),
        in_specs=in_specs,
        out_specs=pl.BlockSpec((Bb, 128), lambda c, s: (0, 0)),
        scratch_shapes=scratch,
    )

    out = pl.pallas_call(
        body,
        grid_spec=grid_spec,
        out_shape=jax.ShapeDtypeStruct((Bb, 128), f32),
        compiler_params=pltpu.CompilerParams(
            dimension_semantics=("arbitrary",),
            vmem_limit_bytes=100 * 1024 * 1024,
        ),
    )(scalars, x, code_b, code_b, idxS, idxO, nS_b, nO_b, *w_args, fcw_pad, fcb_pad)

    return out[:, :C]


# speedup vs baseline: 1.0849x; 1.0849x over previous
import functools

import jax
import jax.numpy as jnp
from jax.experimental import pallas as pl
from jax.experimental.pallas import tpu as pltpu

CT = 16


def _fused_body(Bb, Hh, T, nc,
                s_ref,
                x_ref, code0_ref, code1_ref, idxS_ref, idxO_ref, nS_ref, nO_ref,
                wi1, wh1, bi1, bh1, wi2, wh2, bi2, bh2,
                wis1, whs1, bis1, bhs1, wis2, whs2, bis2, bhs2,
                wio1, who1, bio1, bho1, wio2, who2, bio2, bho2,
                fcw, fcb,
                out_ref,
                g1, g2, g3, g4, gSO, y2,
                y1, ys1, yo1,
                h1, h2, hs1, hs2, ho1, ho2):
    c = pl.program_id(0)
    f32 = jnp.float32
    bf16 = jnp.bfloat16
    ncA = s_ref[0]
    ncB = s_ref[1]
    p = jax.lax.rem(c, 2)
    q = 1 - p
    cb = c - (nc + 1)

    @pl.when(c == 0)
    def _init():
        for r in (h1, h2, hs1, hs2, ho1, ho2, y1, ys1, yo1, y2):
            r[...] = jnp.zeros_like(r)

    def dense(src, w_ref, b_ref, dst_ref):
        Xm = src.reshape(CT * Bb, -1).astype(bf16)
        dst_ref[...] = (
            jnp.dot(Xm, w_ref[...], preferred_element_type=f32) + b_ref[0:1, :]
        ).reshape(CT, Bb, 3 * Hh)

    def cell(gi, gh, h, bhn):
        r = jax.nn.sigmoid(gi[:, :Hh] + gh[:, :Hh])
        z = jax.nn.sigmoid(gi[:, Hh:2 * Hh] + gh[:, Hh:2 * Hh])
        n = jnp.tanh(gi[:, 2 * Hh:] + r * (gh[:, 2 * Hh:] + bhn))
        return (1.0 - z) * n + z * h

    def chain(gi_ref, t, h_ref, w_ref, b_ref, m):
        h = h_ref[...]
        gh = jnp.dot(h.astype(bf16), w_ref[...], preferred_element_type=f32)
        hv = jnp.where(m, cell(gi_ref[t], gh, h, b_ref[0:1, 2 * Hh:]), h)
        h_ref[...] = hv
        return hv

    @pl.when(c <= ncA)
    def _phase_a():
        dense(x_ref[...], wi1, bi1, g1)
        dense(y1[q], wi2, bi2, g2)
        a0 = c < ncA
        a1 = (c >= 1) & (c <= ncA)

        def step(t, carry):
            c0 = code0_ref[t]
            c1 = code1_ref[t]
            y1[p, t] = chain(g1, t, h1, wh1, bh1, (c0 != 0.0) & a0)
            hv2 = chain(g2, t, h2, wh2, bh2, (c1 != 0.0) & a1)
            tg = jnp.maximum((c - 1) * CT + t, 0)
            y2[pl.ds(tg * Bb, Bb), :] = hv2
            return carry

        jax.lax.fori_loop(0, CT, step, 0, unroll=16)

    @pl.when((cb >= 0) & (cb < ncB))
    def _gather():
        iota_b = jax.lax.broadcasted_iota(jnp.int32, (CT, Bb), 1)
        targ = jnp.concatenate(
            [idxS_ref[...] * Bb + iota_b, idxO_ref[...] * Bb + iota_b], axis=0)
        iota_col = jax.lax.broadcasted_iota(jnp.int32, (1, 1, T * Bb), 2)
        p_all = (targ[:, :, None] == iota_col).astype(bf16).reshape(
            2 * CT * Bb, T * Bb)
        res = jnp.dot(p_all.astype(f32), y2[...], preferred_element_type=f32)
        gSO[...] = res.reshape(2, CT, Bb, Hh).astype(bf16)

    @pl.when((cb >= 0) & (cb <= ncB))
    def _phase_b():
        dense(gSO[0], wis1, bis1, g1)
        dense(gSO[1], wio1, bio1, g2)
        dense(ys1[q], wis2, bis2, g3)
        dense(yo1[q], wio2, bio2, g4)
        aL1 = cb < ncB
        aL2 = (cb >= 1) & (cb <= ncB)
        nS = nS_ref[...]
        nO = nO_ref[...]

        def step(t, carry):
            jg = cb * CT + t
            j2 = jg - CT
            jgf = jg.astype(f32)
            j2f = j2.astype(f32)
            ys1[p, t] = chain(g1, t, hs1, whs1, bhs1, (nS > jgf) & aL1)
            yo1[p, t] = chain(g2, t, ho1, who1, bho1, (nO > jgf) & aL1)
            chain(g3, t, hs2, whs2, bhs2, (nS > j2f) & aL2)
            chain(g4, t, ho2, who2, bho2, (nO > j2f) & aL2)
            return carry

        jax.lax.fori_loop(0, CT, step, 0, unroll=16)

    @pl.when(c == 2 * nc + 1)
    def _final():
        zero1 = jnp.zeros((1, Hh), f32)

        zero3 = jnp.zeros((1, 3 * Hh), f32)

        def fall2(bi_1, bh_1, wi_2, bi_2, bh_2):
            f1 = cell(bi_1[0:1, :], zero3, zero1, bh_1[0:1, 2 * Hh:])
            gi = jnp.dot(f1.astype(bf16), wi_2[...], preferred_element_type=f32) + bi_2[0:1, :]
            return cell(gi, zero3, zero1, bh_2[0:1, 2 * Hh:])

        fs = fall2(bis1, bhs1, wis2, bis2, bhs2)
        fo = fall2(bio1, bho1, wio2, bio2, bho2)
        hS = jnp.where(nS_ref[...] > 0.0, hs2[...], fs)
        hO = jnp.where(nO_ref[...] > 0.0, ho2[...], fo)
        hcat = jnp.concatenate([hS, hO, h2[...]], axis=1)
        out_ref[...] = jnp.dot(hcat, fcw[...], preferred_element_type=f32) + fcb[...]


def kernel(context_features, params_inter, params_spk, params_oth, fc_w, fc_b,
           context_lengths, context_speaker_ids, roles):
    f32 = jnp.float32
    bf16 = jnp.bfloat16
    Bb, T, D = context_features.shape
    Hh = params_inter[0][1].shape[1]
    C = fc_w.shape[0]
    nc = T // CT

    x = jnp.transpose(context_features, (1, 0, 2)).astype(bf16)

    lengths = jnp.asarray(context_lengths)
    sid = jnp.asarray(context_speaker_ids)
    roles_a = jnp.asarray(roles)
    t_idx = jnp.arange(T)
    valid = t_idx[:, None] < lengths[None, :]
    match = sid.T == roles_a[None, :]
    spk = valid & match
    oth = valid & (~match)
    code = jnp.where(valid, jnp.where(match, 1.0, -1.0), 0.0).astype(bf16)
    code_b = jnp.broadcast_to(code[:, :, None], (T, Bb, Hh))

    nS = jnp.sum(spk, axis=0)
    nO = jnp.sum(oth, axis=0)
    idxS = jnp.argsort(~spk, axis=0, stable=True).astype(jnp.int32)
    idxO = jnp.argsort(~oth, axis=0, stable=True).astype(jnp.int32)
    maxL = jnp.max(lengths)
    maxSub = jnp.maximum(jnp.max(nS), jnp.max(nO))
    ncA = jnp.clip((maxL + CT - 1) // CT, 1, nc).astype(jnp.int32)
    ncB = jnp.clip((maxSub + CT - 1) // CT, 1, nc).astype(jnp.int32)
    scalars = jnp.stack([ncA, ncB])
    nS_b = jnp.broadcast_to(nS.astype(f32)[:, None], (Bb, Hh))
    nO_b = jnp.broadcast_to(nO.astype(f32)[:, None], (Bb, Hh))

    def prep(pr):
        W_ih, W_hh, b_ih, b_hh = pr
        bi_fold = (b_ih + jnp.concatenate(
            [b_hh[:2 * Hh], jnp.zeros((Hh,), b_hh.dtype)])).astype(f32)
        return (W_ih.T.astype(bf16), W_hh.T.astype(bf16),
                jnp.broadcast_to(bi_fold[None, :], (Bb, 3 * Hh)),
                jnp.broadcast_to(b_hh[None, :].astype(f32), (Bb, 3 * Hh)))

    layers = [prep(pr) for pr in (params_inter + params_spk + params_oth)]
    w_args = [a for lay in layers for a in lay]

    fcw_pad = jnp.zeros((3 * Hh, 128), f32).at[:, :C].set(fc_w.T.astype(f32))
    fcb_pad = jnp.broadcast_to(
        jnp.zeros((128,), f32).at[:C].set(fc_b.astype(f32))[None, :], (Bb, 128))

    def a_spec(k, shape):
        return pl.BlockSpec(
            shape,
            lambda c, s, k=k: (jnp.clip(c - k, 0, jnp.minimum(s[0], nc - 1)), 0, 0))

    def b_spec(shape):
        return pl.BlockSpec(
            shape,
            lambda c, s: (jnp.clip(c - (nc + 1), 0, jnp.minimum(s[1], nc - 1)), 0))

    full2d = lambda a: pl.BlockSpec(a.shape, lambda c, s: (0, 0))
    in_specs = [
        a_spec(0, (CT, Bb, D)),
        a_spec(0, (CT, Bb, Hh)), a_spec(1, (CT, Bb, Hh)),
        b_spec((CT, Bb)), b_spec((CT, Bb)),
        full2d(nS_b), full2d(nO_b),
    ] + [full2d(a) for a in w_args] + [full2d(fcw_pad), full2d(fcb_pad)]

    scratch = (
        [pltpu.VMEM((CT, Bb, 3 * Hh), f32)] * 4
        + [pltpu.VMEM((2, CT, Bb, Hh), bf16)]
        + [pltpu.VMEM((T * Bb, Hh), f32)]
        + [pltpu.VMEM((2, CT, Bb, Hh), f32)] * 3
        + [pltpu.VMEM((Bb, Hh), f32)] * 6
    )

    body = functools.partial(_fused_body, Bb, Hh, T, nc)

    grid_spec = pltpu.PrefetchScalarGridSpec(
        num_scalar_prefetch=1,
        grid=(2 * nc + 2,),
        in_specs=in_specs,
        out_specs=pl.BlockSpec((Bb, 128), lambda c, s: (0, 0)),
        scratch_shapes=scratch,
    )

    out = pl.pallas_call(
        body,
        grid_spec=grid_spec,
        out_shape=jax.ShapeDtypeStruct((Bb, 128), f32),
        compiler_params=pltpu.CompilerParams(
            dimension_semantics=("arbitrary",),
            vmem_limit_bytes=100 * 1024 * 1024,
        ),
    )(scalars, x, code_b, code_b, idxS, idxO, nS_b, nO_b, *w_args, fcw_pad, fcb_pad)

    return out[:, :C]


# speedup vs baseline: 1.0857x; 1.0007x over previous
import functools

import jax
import jax.numpy as jnp
from jax.experimental import pallas as pl
from jax.experimental.pallas import tpu as pltpu

CT = 16


def _fused_body(Bb, Hh, T, nc,
                s_ref,
                x_ref, code0_ref, code1_ref, idxS_ref, idxO_ref, nS_ref, nO_ref,
                wi1, wh1, bi1, bh1, wi2, wh2, bi2, bh2,
                wis1, whs1, bis1, bhs1, wis2, whs2, bis2, bhs2,
                wio1, who1, bio1, bho1, wio2, who2, bio2, bho2,
                fcw, fcb,
                out_ref,
                g1, g2, g3, g4, gSO, y2,
                y1, ys1, yo1,
                h1, h2, hs1, hs2, ho1, ho2):
    c = pl.program_id(0)
    f32 = jnp.float32
    bf16 = jnp.bfloat16
    ncA = s_ref[0]
    ncB = s_ref[1]
    p = jax.lax.rem(c, 2)
    q = 1 - p
    cb = c - (nc + 1)

    @pl.when(c == 0)
    def _init():
        for r in (h1, h2, hs1, hs2, ho1, ho2, y1, ys1, yo1, y2):
            r[...] = jnp.zeros_like(r)

    def dense(src, w_ref, b_ref, dst_ref):
        Xm = src.reshape(CT * Bb, -1).astype(bf16)
        dst_ref[...] = (
            jnp.dot(Xm, w_ref[...], preferred_element_type=f32) + b_ref[0:1, :]
        ).reshape(CT, Bb, 3 * Hh)

    def cell(gi, gh, h, bhn):
        r = jax.nn.sigmoid(gi[:, :Hh] + gh[:, :Hh])
        z = jax.nn.sigmoid(gi[:, Hh:2 * Hh] + gh[:, Hh:2 * Hh])
        n = jnp.tanh(gi[:, 2 * Hh:] + r * (gh[:, 2 * Hh:] + bhn))
        return (1.0 - z) * n + z * h

    def chain(gi_ref, t, h_ref, w_ref, b_ref, m):
        h = h_ref[...]
        hb = h.astype(bf16)
        gi = gi_ref[t]
        ghrz = jnp.dot(hb, w_ref[:, :2 * Hh], preferred_element_type=f32)
        r = jax.nn.sigmoid(gi[:, :Hh] + ghrz[:, :Hh])
        z = jax.nn.sigmoid(gi[:, Hh:2 * Hh] + ghrz[:, Hh:])
        ghn = jnp.dot(hb, w_ref[:, 2 * Hh:], preferred_element_type=f32)
        n = jnp.tanh(gi[:, 2 * Hh:] + r * (ghn + b_ref[0:1, 2 * Hh:]))
        hv = jnp.where(m, (1.0 - z) * n + z * h, h)
        h_ref[...] = hv
        return hv

    @pl.when(c <= ncA)
    def _phase_a():
        dense(x_ref[...], wi1, bi1, g1)
        dense(y1[q], wi2, bi2, g2)
        a0 = c < ncA
        a1 = (c >= 1) & (c <= ncA)

        def step(t, carry):
            c0 = code0_ref[t]
            c1 = code1_ref[t]
            y1[p, t] = chain(g1, t, h1, wh1, bh1, (c0 != 0.0) & a0)
            hv2 = chain(g2, t, h2, wh2, bh2, (c1 != 0.0) & a1)
            tg = jnp.maximum((c - 1) * CT + t, 0)
            y2[pl.ds(tg * Bb, Bb), :] = hv2
            return carry

        jax.lax.fori_loop(0, CT, step, 0, unroll=16)

    @pl.when((cb >= 0) & (cb < ncB))
    def _gather():
        iota_b = jax.lax.broadcasted_iota(jnp.int32, (CT, Bb), 1)
        targ = jnp.concatenate(
            [idxS_ref[...] * Bb + iota_b, idxO_ref[...] * Bb + iota_b], axis=0)
        iota_col = jax.lax.broadcasted_iota(jnp.int32, (1, 1, T * Bb), 2)
        p_all = (targ[:, :, None] == iota_col).astype(bf16).reshape(
            2 * CT * Bb, T * Bb)
        res = jnp.dot(p_all.astype(f32), y2[...], preferred_element_type=f32)
        gSO[...] = res.reshape(2, CT, Bb, Hh).astype(bf16)

    @pl.when((cb >= 0) & (cb <= ncB))
    def _phase_b():
        dense(gSO[0], wis1, bis1, g1)
        dense(gSO[1], wio1, bio1, g2)
        dense(ys1[q], wis2, bis2, g3)
        dense(yo1[q], wio2, bio2, g4)
        aL1 = cb < ncB
        aL2 = (cb >= 1) & (cb <= ncB)
        nS = nS_ref[...]
        nO = nO_ref[...]

        def step(t, carry):
            jg = cb * CT + t
            j2 = jg - CT
            jgf = jg.astype(f32)
            j2f = j2.astype(f32)
            ys1[p, t] = chain(g1, t, hs1, whs1, bhs1, (nS > jgf) & aL1)
            yo1[p, t] = chain(g2, t, ho1, who1, bho1, (nO > jgf) & aL1)
            chain(g3, t, hs2, whs2, bhs2, (nS > j2f) & aL2)
            chain(g4, t, ho2, who2, bho2, (nO > j2f) & aL2)
            return carry

        jax.lax.fori_loop(0, CT, step, 0, unroll=16)

    @pl.when(c == 2 * nc + 1)
    def _final():
        zero1 = jnp.zeros((1, Hh), f32)

        zero3 = jnp.zeros((1, 3 * Hh), f32)

        def fall2(bi_1, bh_1, wi_2, bi_2, bh_2):
            f1 = cell(bi_1[0:1, :], zero3, zero1, bh_1[0:1, 2 * Hh:])
            gi = jnp.dot(f1.astype(bf16), wi_2[...], preferred_element_type=f32) + bi_2[0:1, :]
            return cell(gi, zero3, zero1, bh_2[0:1, 2 * Hh:])

        fs = fall2(bis1, bhs1, wis2, bis2, bhs2)
        fo = fall2(bio1, bho1, wio2, bio2, bho2)
        hS = jnp.where(nS_ref[...] > 0.0, hs2[...], fs)
        hO = jnp.where(nO_ref[...] > 0.0, ho2[...], fo)
        hcat = jnp.concatenate([hS, hO, h2[...]], axis=1)
        out_ref[...] = jnp.dot(hcat, fcw[...], preferred_element_type=f32) + fcb[...]


def kernel(context_features, params_inter, params_spk, params_oth, fc_w, fc_b,
           context_lengths, context_speaker_ids, roles):
    f32 = jnp.float32
    bf16 = jnp.bfloat16
    Bb, T, D = context_features.shape
    Hh = params_inter[0][1].shape[1]
    C = fc_w.shape[0]
    nc = T // CT

    x = jnp.transpose(context_features, (1, 0, 2)).astype(bf16)

    lengths = jnp.asarray(context_lengths)
    sid = jnp.asarray(context_speaker_ids)
    roles_a = jnp.asarray(roles)
    t_idx = jnp.arange(T)
    valid = t_idx[:, None] < lengths[None, :]
    match = sid.T == roles_a[None, :]
    spk = valid & match
    oth = valid & (~match)
    code = jnp.where(valid, jnp.where(match, 1.0, -1.0), 0.0).astype(bf16)
    code_b = jnp.broadcast_to(code[:, :, None], (T, Bb, Hh))

    nS = jnp.sum(spk, axis=0)
    nO = jnp.sum(oth, axis=0)
    idxS = jnp.argsort(~spk, axis=0, stable=True).astype(jnp.int32)
    idxO = jnp.argsort(~oth, axis=0, stable=True).astype(jnp.int32)
    maxL = jnp.max(lengths)
    maxSub = jnp.maximum(jnp.max(nS), jnp.max(nO))
    ncA = jnp.clip((maxL + CT - 1) // CT, 1, nc).astype(jnp.int32)
    ncB = jnp.clip((maxSub + CT - 1) // CT, 1, nc).astype(jnp.int32)
    scalars = jnp.stack([ncA, ncB])
    nS_b = jnp.broadcast_to(nS.astype(f32)[:, None], (Bb, Hh))
    nO_b = jnp.broadcast_to(nO.astype(f32)[:, None], (Bb, Hh))

    def prep(pr):
        W_ih, W_hh, b_ih, b_hh = pr
        bi_fold = (b_ih + jnp.concatenate(
            [b_hh[:2 * Hh], jnp.zeros((Hh,), b_hh.dtype)])).astype(f32)
        return (W_ih.T.astype(bf16), W_hh.T.astype(bf16),
                jnp.broadcast_to(bi_fold[None, :], (Bb, 3 * Hh)),
                jnp.broadcast_to(b_hh[None, :].astype(f32), (Bb, 3 * Hh)))

    layers = [prep(pr) for pr in (params_inter + params_spk + params_oth)]
    w_args = [a for lay in layers for a in lay]

    fcw_pad = jnp.zeros((3 * Hh, 128), f32).at[:, :C].set(fc_w.T.astype(f32))
    fcb_pad = jnp.broadcast_to(
        jnp.zeros((128,), f32).at[:C].set(fc_b.astype(f32))[None, :], (Bb, 128))

    def a_spec(k, shape):
        return pl.BlockSpec(
            shape,
            lambda c, s, k=k: (jnp.clip(c - k, 0, jnp.minimum(s[0], nc - 1)), 0, 0))

    def b_spec(shape):
        return pl.BlockSpec(
            shape,
            lambda c, s: (jnp.clip(c - (nc + 1), 0, jnp.minimum(s[1], nc - 1)), 0))

    full2d = lambda a: pl.BlockSpec(a.shape, lambda c, s: (0, 0))
    in_specs = [
        a_spec(0, (CT, Bb, D)),
        a_spec(0, (CT, Bb, Hh)), a_spec(1, (CT, Bb, Hh)),
        b_spec((CT, Bb)), b_spec((CT, Bb)),
        full2d(nS_b), full2d(nO_b),
    ] + [full2d(a) for a in w_args] + [full2d(fcw_pad), full2d(fcb_pad)]

    scratch = (
        [pltpu.VMEM((CT, Bb, 3 * Hh), f32)] * 4
        + [pltpu.VMEM((2, CT, Bb, Hh), bf16)]
        + [pltpu.VMEM((T * Bb, Hh), f32)]
        + [pltpu.VMEM((2, CT, Bb, Hh), f32)] * 3
        + [pltpu.VMEM((Bb, Hh), f32)] * 6
    )

    body = functools.partial(_fused_body, Bb, Hh, T, nc)

    grid_spec = pltpu.PrefetchScalarGridSpec(
        num_scalar_prefetch=1,
        grid=(2 * nc + 2,),
        in_specs=in_specs,
        out_specs=pl.BlockSpec((Bb, 128), lambda c, s: (0, 0)),
        scratch_shapes=scratch,
    )

    out = pl.pallas_call(
        body,
        grid_spec=grid_spec,
        out_shape=jax.ShapeDtypeStruct((Bb, 128), f32),
        compiler_params=pltpu.CompilerParams(
            dimension_semantics=("arbitrary",),
            vmem_limit_bytes=100 * 1024 * 1024,
        ),
    )(scalars, x, code_b, code_b, idxS, idxO, nS_b, nO_b, *w_args, fcw_pad, fcb_pad)

    return out[:, :C]


# speedup vs baseline: 1.0870x; 1.0013x over previous
import functools

import jax
import jax.numpy as jnp
from jax.experimental import pallas as pl
from jax.experimental.pallas import tpu as pltpu

CT = 16


def _fused_body(Bb, Hh, T, nc,
                s_ref,
                x_ref, code0_ref, code1_ref, idxS_ref, idxO_ref, nS_ref, nO_ref,
                wi1, wh1, bi1, bh1, wi2, wh2, bi2, bh2,
                wis1, whs1, bis1, bhs1, wis2, whs2, bis2, bhs2,
                wio1, who1, bio1, bho1, wio2, who2, bio2, bho2,
                fcw, fcb,
                out_ref,
                g1, g2, g3, g4, gSO, y2,
                y1, ys1, yo1,
                h1, h2, hs1, hs2, ho1, ho2):
    c = pl.program_id(0)
    f32 = jnp.float32
    bf16 = jnp.bfloat16
    ncA = s_ref[0]
    ncB = s_ref[1]
    p = jax.lax.rem(c, 2)
    q = 1 - p
    cb = c - (nc + 1)

    @pl.when(c == 0)
    def _init():
        for r in (h1, h2, hs1, hs2, ho1, ho2, y1, ys1, yo1, y2):
            r[...] = jnp.zeros_like(r)

    def dense(src, w_ref, b_ref, dst_ref):
        Xm = src.reshape(CT * Bb, -1).astype(bf16)
        dst_ref[...] = (
            jnp.dot(Xm, w_ref[...], preferred_element_type=f32) + b_ref[0:1, :]
        ).reshape(CT, Bb, 3 * Hh)

    def cell(gi, gh, h, bhn):
        r = jax.nn.sigmoid(gi[:, :Hh] + gh[:, :Hh])
        z = jax.nn.sigmoid(gi[:, Hh:2 * Hh] + gh[:, Hh:2 * Hh])
        n = jnp.tanh(gi[:, 2 * Hh:] + r * (gh[:, 2 * Hh:] + bhn))
        return (1.0 - z) * n + z * h

    def chain(gi_ref, t, h_ref, w_ref, b_ref, m):
        h = h_ref[...]
        hb = h.astype(bf16)
        gi = gi_ref[t]
        ghrz = jnp.dot(hb, w_ref[:, :2 * Hh], preferred_element_type=f32)
        r = jax.nn.sigmoid(gi[:, :Hh] + ghrz[:, :Hh])
        z = jax.nn.sigmoid(gi[:, Hh:2 * Hh] + ghrz[:, Hh:])
        ghn = jnp.dot(hb, w_ref[:, 2 * Hh:], preferred_element_type=f32)
        n = jnp.tanh(gi[:, 2 * Hh:] + r * (ghn + b_ref[0:1, 2 * Hh:]))
        hv = jnp.where(m, (1.0 - z) * n + z * h, h)
        h_ref[...] = hv
        return hv

    @pl.when(c <= ncA)
    def _phase_a():
        dense(x_ref[...], wi1, bi1, g1)
        dense(y1[q], wi2, bi2, g2)
        a0 = c < ncA
        a1 = (c >= 1) & (c <= ncA)

        def step(t, carry):
            c0 = code0_ref[t]
            c1 = code1_ref[t]
            y1[p, t] = chain(g1, t, h1, wh1, bh1, (c0 != 0.0) & a0).astype(bf16)
            hv2 = chain(g2, t, h2, wh2, bh2, (c1 != 0.0) & a1)
            tg = jnp.maximum((c - 1) * CT + t, 0)
            y2[pl.ds(tg * Bb, Bb), :] = hv2.astype(bf16)
            return carry

        jax.lax.fori_loop(0, CT, step, 0, unroll=16)

    @pl.when((cb >= 0) & (cb < ncB))
    def _gather():
        iota_b = jax.lax.broadcasted_iota(jnp.int32, (CT, Bb), 1)
        targ = jnp.concatenate(
            [idxS_ref[...] * Bb + iota_b, idxO_ref[...] * Bb + iota_b], axis=0)
        iota_col = jax.lax.broadcasted_iota(jnp.int32, (1, 1, T * Bb), 2)
        p_all = (targ[:, :, None] == iota_col).astype(bf16).reshape(
            2 * CT * Bb, T * Bb)
        res = jnp.dot(p_all, y2[...], preferred_element_type=f32)
        gSO[...] = res.reshape(2, CT, Bb, Hh).astype(bf16)

    @pl.when((cb >= 0) & (cb <= ncB))
    def _phase_b():
        dense(gSO[0], wis1, bis1, g1)
        dense(gSO[1], wio1, bio1, g2)
        dense(ys1[q], wis2, bis2, g3)
        dense(yo1[q], wio2, bio2, g4)
        aL1 = cb < ncB
        aL2 = (cb >= 1) & (cb <= ncB)
        nS = nS_ref[...]
        nO = nO_ref[...]

        def step(t, carry):
            jg = cb * CT + t
            j2 = jg - CT
            jgf = jg.astype(f32)
            j2f = j2.astype(f32)
            ys1[p, t] = chain(g1, t, hs1, whs1, bhs1, (nS > jgf) & aL1).astype(bf16)
            yo1[p, t] = chain(g2, t, ho1, who1, bho1, (nO > jgf) & aL1).astype(bf16)
            chain(g3, t, hs2, whs2, bhs2, (nS > j2f) & aL2)
            chain(g4, t, ho2, who2, bho2, (nO > j2f) & aL2)
            return carry

        jax.lax.fori_loop(0, CT, step, 0, unroll=16)

    @pl.when(c == 2 * nc + 1)
    def _final():
        zero1 = jnp.zeros((1, Hh), f32)

        zero3 = jnp.zeros((1, 3 * Hh), f32)

        def fall2(bi_1, bh_1, wi_2, bi_2, bh_2):
            f1 = cell(bi_1[0:1, :], zero3, zero1, bh_1[0:1, 2 * Hh:])
            gi = jnp.dot(f1.astype(bf16), wi_2[...], preferred_element_type=f32) + bi_2[0:1, :]
            return cell(gi, zero3, zero1, bh_2[0:1, 2 * Hh:])

        fs = fall2(bis1, bhs1, wis2, bis2, bhs2)
        fo = fall2(bio1, bho1, wio2, bio2, bho2)
        hS = jnp.where(nS_ref[...] > 0.0, hs2[...], fs)
        hO = jnp.where(nO_ref[...] > 0.0, ho2[...], fo)
        hcat = jnp.concatenate([hS, hO, h2[...]], axis=1)
        out_ref[...] = jnp.dot(hcat, fcw[...], preferred_element_type=f32) + fcb[...]


def kernel(context_features, params_inter, params_spk, params_oth, fc_w, fc_b,
           context_lengths, context_speaker_ids, roles):
    f32 = jnp.float32
    bf16 = jnp.bfloat16
    Bb, T, D = context_features.shape
    Hh = params_inter[0][1].shape[1]
    C = fc_w.shape[0]
    nc = T // CT

    x = jnp.transpose(context_features, (1, 0, 2)).astype(bf16)

    lengths = jnp.asarray(context_lengths)
    sid = jnp.asarray(context_speaker_ids)
    roles_a = jnp.asarray(roles)
    t_idx = jnp.arange(T)
    valid = t_idx[:, None] < lengths[None, :]
    match = sid.T == roles_a[None, :]
    spk = valid & match
    oth = valid & (~match)
    code = jnp.where(valid, jnp.where(match, 1.0, -1.0), 0.0).astype(bf16)
    code_b = jnp.broadcast_to(code[:, :, None], (T, Bb, Hh))

    nS = jnp.sum(spk, axis=0)
    nO = jnp.sum(oth, axis=0)
    idxS = jnp.argsort(~spk, axis=0, stable=True).astype(jnp.int32)
    idxO = jnp.argsort(~oth, axis=0, stable=True).astype(jnp.int32)
    maxL = jnp.max(lengths)
    maxSub = jnp.maximum(jnp.max(nS), jnp.max(nO))
    ncA = jnp.clip((maxL + CT - 1) // CT, 1, nc).astype(jnp.int32)
    ncB = jnp.clip((maxSub + CT - 1) // CT, 1, nc).astype(jnp.int32)
    scalars = jnp.stack([ncA, ncB])
    nS_b = jnp.broadcast_to(nS.astype(f32)[:, None], (Bb, Hh))
    nO_b = jnp.broadcast_to(nO.astype(f32)[:, None], (Bb, Hh))

    def prep(pr):
        W_ih, W_hh, b_ih, b_hh = pr
        bi_fold = (b_ih + jnp.concatenate(
            [b_hh[:2 * Hh], jnp.zeros((Hh,), b_hh.dtype)])).astype(f32)
        return (W_ih.T.astype(bf16), W_hh.T.astype(bf16),
                jnp.broadcast_to(bi_fold[None, :], (Bb, 3 * Hh)),
                jnp.broadcast_to(b_hh[None, :].astype(f32), (Bb, 3 * Hh)))

    layers = [prep(pr) for pr in (params_inter + params_spk + params_oth)]
    w_args = [a for lay in layers for a in lay]

    fcw_pad = jnp.zeros((3 * Hh, 128), f32).at[:, :C].set(fc_w.T.astype(f32))
    fcb_pad = jnp.broadcast_to(
        jnp.zeros((128,), f32).at[:C].set(fc_b.astype(f32))[None, :], (Bb, 128))

    def a_spec(k, shape):
        return pl.BlockSpec(
            shape,
            lambda c, s, k=k: (jnp.clip(c - k, 0, jnp.minimum(s[0], nc - 1)), 0, 0))

    def b_spec(shape):
        return pl.BlockSpec(
            shape,
            lambda c, s: (jnp.clip(c - (nc + 1), 0, jnp.minimum(s[1], nc - 1)), 0))

    full2d = lambda a: pl.BlockSpec(a.shape, lambda c, s: (0, 0))
    in_specs = [
        a_spec(0, (CT, Bb, D)),
        a_spec(0, (CT, Bb, Hh)), a_spec(1, (CT, Bb, Hh)),
        b_spec((CT, Bb)), b_spec((CT, Bb)),
        full2d(nS_b), full2d(nO_b),
    ] + [full2d(a) for a in w_args] + [full2d(fcw_pad), full2d(fcb_pad)]

    scratch = (
        [pltpu.VMEM((CT, Bb, 3 * Hh), f32)] * 4
        + [pltpu.VMEM((2, CT, Bb, Hh), bf16)]
        + [pltpu.VMEM((T * Bb, Hh), bf16)]
        + [pltpu.VMEM((2, CT, Bb, Hh), bf16)] * 3
        + [pltpu.VMEM((Bb, Hh), f32)] * 6
    )

    body = functools.partial(_fused_body, Bb, Hh, T, nc)

    grid_spec = pltpu.PrefetchScalarGridSpec(
        num_scalar_prefetch=1,
        grid=(2 * nc + 2,),
        in_specs=in_specs,
        out_specs=pl.BlockSpec((Bb, 128), lambda c, s: (0, 0)),
        scratch_shapes=scratch,
    )

    out = pl.pallas_call(
        body,
        grid_spec=grid_spec,
        out_shape=jax.ShapeDtypeStruct((Bb, 128), f32),
        compiler_params=pltpu.CompilerParams(
            dimension_semantics=("arbitrary",),
            vmem_limit_bytes=100 * 1024 * 1024,
        ),
    )(scalars, x, code_b, code_b, idxS, idxO, nS_b, nO_b, *w_args, fcw_pad, fcb_pad)

    return out[:, :C]


# speedup vs baseline: 1.0878x; 1.0007x over previous
import functools

import jax
import jax.numpy as jnp
from jax.experimental import pallas as pl
from jax.experimental.pallas import tpu as pltpu

CT = 16


def _fused_body(Bb, Hh, T, nc,
                s_ref,
                x_ref, code0_ref, code1_ref, idxS_ref, idxO_ref, nS_ref, nO_ref,
                wi1, wh1, bi1, bh1, wi2, wh2, bi2, bh2,
                wis1, whs1, bis1, bhs1, wis2, whs2, bis2, bhs2,
                wio1, who1, bio1, bho1, wio2, who2, bio2, bho2,
                fcw, fcb,
                out_ref,
                g1, g2, g3, g4, gSO, y2,
                y1, ys1, yo1,
                h1, h2, hs1, hs2, ho1, ho2):
    c = pl.program_id(0)
    f32 = jnp.float32
    bf16 = jnp.bfloat16
    ncA = s_ref[0]
    ncB = s_ref[1]
    p = jax.lax.rem(c, 2)
    q = 1 - p
    cb = c - (nc + 1)

    @pl.when(c == 0)
    def _init():
        for r in (h1, h2, hs1, hs2, ho1, ho2, y1, ys1, yo1, y2):
            r[...] = jnp.zeros_like(r)

    def dense(src, w_ref, b_ref, dst_ref):
        Xm = src.reshape(CT * Bb, -1).astype(bf16)
        dst_ref[...] = (
            jnp.dot(Xm, w_ref[...], preferred_element_type=f32) + b_ref[0:1, :]
        ).reshape(CT, Bb, 3 * Hh)

    def cell(gi, gh, h, bhn):
        r = jax.nn.sigmoid(gi[:, :Hh] + gh[:, :Hh])
        z = jax.nn.sigmoid(gi[:, Hh:2 * Hh] + gh[:, Hh:2 * Hh])
        n = jnp.tanh(gi[:, 2 * Hh:] + r * (gh[:, 2 * Hh:] + bhn))
        return (1.0 - z) * n + z * h

    def chain(gi_ref, t, h, w_ref, b_ref, m):
        hb = h.astype(bf16)
        gi = gi_ref[t]
        ghrz = jnp.dot(hb, w_ref[:, :2 * Hh], preferred_element_type=f32)
        r = jax.nn.sigmoid(gi[:, :Hh] + ghrz[:, :Hh])
        z = jax.nn.sigmoid(gi[:, Hh:2 * Hh] + ghrz[:, Hh:])
        ghn = jnp.dot(hb, w_ref[:, 2 * Hh:], preferred_element_type=f32)
        n = jnp.tanh(gi[:, 2 * Hh:] + r * (ghn + b_ref[0:1, 2 * Hh:]))
        return jnp.where(m, (1.0 - z) * n + z * h, h)

    @pl.when(c <= ncA)
    def _phase_a():
        dense(x_ref[...], wi1, bi1, g1)
        dense(y1[q], wi2, bi2, g2)
        a0 = c < ncA
        a1 = (c >= 1) & (c <= ncA)

        def step(t, carry):
            h1v, h2v = carry
            c0 = code0_ref[t]
            c1 = code1_ref[t]
            h1v = chain(g1, t, h1v, wh1, bh1, (c0 != 0.0) & a0)
            y1[p, t] = h1v.astype(bf16)
            h2v = chain(g2, t, h2v, wh2, bh2, (c1 != 0.0) & a1)
            tg = jnp.maximum((c - 1) * CT + t, 0)
            y2[pl.ds(tg * Bb, Bb), :] = h2v.astype(bf16)
            return h1v, h2v

        h1[...], h2[...] = jax.lax.fori_loop(
            0, CT, step, (h1[...], h2[...]), unroll=16)

    @pl.when((cb >= 0) & (cb < ncB))
    def _gather():
        iota_b = jax.lax.broadcasted_iota(jnp.int32, (CT, Bb), 1)
        targ = jnp.concatenate(
            [idxS_ref[...] * Bb + iota_b, idxO_ref[...] * Bb + iota_b], axis=0)
        iota_col = jax.lax.broadcasted_iota(jnp.int32, (1, 1, T * Bb), 2)
        p_all = (targ[:, :, None] == iota_col).astype(bf16).reshape(
            2 * CT * Bb, T * Bb)
        res = jnp.dot(p_all, y2[...], preferred_element_type=f32)
        gSO[...] = res.reshape(2, CT, Bb, Hh).astype(bf16)

    @pl.when((cb >= 0) & (cb <= ncB))
    def _phase_b():
        dense(gSO[0], wis1, bis1, g1)
        dense(gSO[1], wio1, bio1, g2)
        dense(ys1[q], wis2, bis2, g3)
        dense(yo1[q], wio2, bio2, g4)
        aL1 = cb < ncB
        aL2 = (cb >= 1) & (cb <= ncB)
        nS = nS_ref[...]
        nO = nO_ref[...]

        def step(t, carry):
            hs1v, ho1v, hs2v, ho2v = carry
            jg = cb * CT + t
            j2 = jg - CT
            jgf = jg.astype(f32)
            j2f = j2.astype(f32)
            hs1v = chain(g1, t, hs1v, whs1, bhs1, (nS > jgf) & aL1)
            ys1[p, t] = hs1v.astype(bf16)
            ho1v = chain(g2, t, ho1v, who1, bho1, (nO > jgf) & aL1)
            yo1[p, t] = ho1v.astype(bf16)
            hs2v = chain(g3, t, hs2v, whs2, bhs2, (nS > j2f) & aL2)
            ho2v = chain(g4, t, ho2v, who2, bho2, (nO > j2f) & aL2)
            return hs1v, ho1v, hs2v, ho2v

        hs1[...], ho1[...], hs2[...], ho2[...] = jax.lax.fori_loop(
            0, CT, step, (hs1[...], ho1[...], hs2[...], ho2[...]), unroll=16)

    @pl.when(c == 2 * nc + 1)
    def _final():
        zero1 = jnp.zeros((1, Hh), f32)

        zero3 = jnp.zeros((1, 3 * Hh), f32)

        def fall2(bi_1, bh_1, wi_2, bi_2, bh_2):
            f1 = cell(bi_1[0:1, :], zero3, zero1, bh_1[0:1, 2 * Hh:])
            gi = jnp.dot(f1.astype(bf16), wi_2[...], preferred_element_type=f32) + bi_2[0:1, :]
            return cell(gi, zero3, zero1, bh_2[0:1, 2 * Hh:])

        fs = fall2(bis1, bhs1, wis2, bis2, bhs2)
        fo = fall2(bio1, bho1, wio2, bio2, bho2)
        hS = jnp.where(nS_ref[...] > 0.0, hs2[...], fs)
        hO = jnp.where(nO_ref[...] > 0.0, ho2[...], fo)
        hcat = jnp.concatenate([hS, hO, h2[...]], axis=1)
        out_ref[...] = jnp.dot(hcat, fcw[...], preferred_element_type=f32) + fcb[...]


def kernel(context_features, params_inter, params_spk, params_oth, fc_w, fc_b,
           context_lengths, context_speaker_ids, roles):
    f32 = jnp.float32
    bf16 = jnp.bfloat16
    Bb, T, D = context_features.shape
    Hh = params_inter[0][1].shape[1]
    C = fc_w.shape[0]
    nc = T // CT

    x = jnp.transpose(context_features, (1, 0, 2)).astype(bf16)

    lengths = jnp.asarray(context_lengths)
    sid = jnp.asarray(context_speaker_ids)
    roles_a = jnp.asarray(roles)
    t_idx = jnp.arange(T)
    valid = t_idx[:, None] < lengths[None, :]
    match = sid.T == roles_a[None, :]
    spk = valid & match
    oth = valid & (~match)
    code = jnp.where(valid, jnp.where(match, 1.0, -1.0), 0.0).astype(bf16)
    code_b = jnp.broadcast_to(code[:, :, None], (T, Bb, Hh))

    nS = jnp.sum(spk, axis=0)
    nO = jnp.sum(oth, axis=0)
    idxS = jnp.argsort(~spk, axis=0, stable=True).astype(jnp.int32)
    idxO = jnp.argsort(~oth, axis=0, stable=True).astype(jnp.int32)
    maxL = jnp.max(lengths)
    maxSub = jnp.maximum(jnp.max(nS), jnp.max(nO))
    ncA = jnp.clip((maxL + CT - 1) // CT, 1, nc).astype(jnp.int32)
    ncB = jnp.clip((maxSub + CT - 1) // CT, 1, nc).astype(jnp.int32)
    scalars = jnp.stack([ncA, ncB])
    nS_b = jnp.broadcast_to(nS.astype(f32)[:, None], (Bb, Hh))
    nO_b = jnp.broadcast_to(nO.astype(f32)[:, None], (Bb, Hh))

    def prep(pr):
        W_ih, W_hh, b_ih, b_hh = pr
        bi_fold = (b_ih + jnp.concatenate(
            [b_hh[:2 * Hh], jnp.zeros((Hh,), b_hh.dtype)])).astype(f32)
        return (W_ih.T.astype(bf16), W_hh.T.astype(bf16),
                jnp.broadcast_to(bi_fold[None, :], (Bb, 3 * Hh)),
                jnp.broadcast_to(b_hh[None, :].astype(f32), (Bb, 3 * Hh)))

    layers = [prep(pr) for pr in (params_inter + params_spk + params_oth)]
    w_args = [a for lay in layers for a in lay]

    fcw_pad = jnp.zeros((3 * Hh, 128), f32).at[:, :C].set(fc_w.T.astype(f32))
    fcb_pad = jnp.broadcast_to(
        jnp.zeros((128,), f32).at[:C].set(fc_b.astype(f32))[None, :], (Bb, 128))

    def a_spec(k, shape):
        return pl.BlockSpec(
            shape,
            lambda c, s, k=k: (jnp.clip(c - k, 0, jnp.minimum(s[0], nc - 1)), 0, 0))

    def b_spec(shape):
        return pl.BlockSpec(
            shape,
            lambda c, s: (jnp.clip(c - (nc + 1), 0, jnp.minimum(s[1], nc - 1)), 0))

    full2d = lambda a: pl.BlockSpec(a.shape, lambda c, s: (0, 0))
    in_specs = [
        a_spec(0, (CT, Bb, D)),
        a_spec(0, (CT, Bb, Hh)), a_spec(1, (CT, Bb, Hh)),
        b_spec((CT, Bb)), b_spec((CT, Bb)),
        full2d(nS_b), full2d(nO_b),
    ] + [full2d(a) for a in w_args] + [full2d(fcw_pad), full2d(fcb_pad)]

    scratch = (
        [pltpu.VMEM((CT, Bb, 3 * Hh), f32)] * 4
        + [pltpu.VMEM((2, CT, Bb, Hh), bf16)]
        + [pltpu.VMEM((T * Bb, Hh), bf16)]
        + [pltpu.VMEM((2, CT, Bb, Hh), bf16)] * 3
        + [pltpu.VMEM((Bb, Hh), f32)] * 6
    )

    body = functools.partial(_fused_body, Bb, Hh, T, nc)

    grid_spec = pltpu.PrefetchScalarGridSpec(
        num_scalar_prefetch=1,
        grid=(2 * nc + 2,),
        in_specs=in_specs,
        out_specs=pl.BlockSpec((Bb, 128), lambda c, s: (0, 0)),
        scratch_shapes=scratch,
    )

    out = pl.pallas_call(
        body,
        grid_spec=grid_spec,
        out_shape=jax.ShapeDtypeStruct((Bb, 128), f32),
        compiler_params=pltpu.CompilerParams(
            dimension_semantics=("arbitrary",),
            vmem_limit_bytes=100 * 1024 * 1024,
        ),
    )(scalars, x, code_b, code_b, idxS, idxO, nS_b, nO_b, *w_args, fcw_pad, fcb_pad)

    return out[:, :C]
